# baseline (speedup 1.0000x reference)
import jax
import jax.numpy as jnp
from jax.experimental import pallas as pl
from jax.experimental.pallas import tpu as pltpu

_B = 3
_S = 2048
_E = 1024
_HD = 64
_BS = 16
_CHUNK = 512
_NBUF = 6
_SCALE = _HD ** -0.5


def _paged_attn_kernel(x_hbm, wq_ref, wk_ref, wv_ref, out_ref,
                       xbuf, x16buf, sem, sem16):
    nchunks = _B * _S // _CHUNK
    chunks_per_req = _S // _CHUNK

    def copy(c):
        return pltpu.make_async_copy(
            x_hbm.at[pl.ds(c * _CHUNK, _CHUNK), :], xbuf.at[c % _NBUF],
            sem.at[c % _NBUF])

    def copy16(b):
        return pltpu.make_async_copy(
            x_hbm.at[pl.ds(b * _S, _BS), :], x16buf.at[b], sem16.at[b])

    for b in range(_B):
        copy16(b).start()
    for c in range(_NBUF):
        copy(c).start()

    dn_nt = (((1,), (1,)), ((), ()))
    dn_nn = (((1,), (0,)), ((), ()))

    a_req = [None] * _B
    v_req = [None] * _B
    for c in range(nchunks):
        b = c // chunks_per_req
        if c % chunks_per_req == 0:
            copy16(b).wait()
            x16 = x16buf[b]
            k16 = jax.lax.dot_general(x16, wk_ref[:, :], dn_nt,
                                      preferred_element_type=jnp.float32)
            v_req[b] = jax.lax.dot_general(x16, wv_ref[:, :], dn_nt,
                                           preferred_element_type=jnp.float32)
            a_req[b] = jax.lax.dot_general(k16 * _SCALE, wq_ref[:, :], dn_nn,
                                           preferred_element_type=jnp.float32)

        copy(c).wait()
        x_tile = xbuf[c % _NBUF]

        s_t = jax.lax.dot_general(a_req[b], x_tile, dn_nt,
                                  preferred_element_type=jnp.float32)

        row = jax.lax.broadcasted_iota(jnp.int32, (_BS, _CHUNK), 1)
        m = jax.lax.broadcasted_iota(jnp.int32, (_BS, _CHUNK), 0)
        d = (c % chunks_per_req) * (_CHUNK // _BS) + (row >> 4)
        r = row & (_BS - 1)
        cnt = d.astype(jnp.float32) + (m <= r).astype(jnp.float32)

        smax = jnp.max(s_t, axis=0, keepdims=True)
        w = cnt * jnp.exp(s_t - smax)
        w = w / jnp.sum(w, axis=0, keepdims=True)

        out = jax.lax.dot_general(w, v_req[b],
                                  (((0,), (0,)), ((), ())),
                                  preferred_element_type=jnp.float32)
        out_ref[pl.ds(c * _CHUNK, _CHUNK), :] = out

        if c + _NBUF < nchunks:
            copy(c + _NBUF).start()


@jax.jit
def kernel(x, Wq, Wk, Wv):
    xf = x.reshape(_B * _S, _E)
    out = pl.pallas_call(
        _paged_attn_kernel,
        in_specs=[
            pl.BlockSpec(memory_space=pltpu.MemorySpace.HBM),
            pl.BlockSpec((_HD, _E), lambda: (0, 0)),
            pl.BlockSpec((_HD, _E), lambda: (0, 0)),
            pl.BlockSpec((_HD, _E), lambda: (0, 0)),
        ],
        out_specs=pl.BlockSpec((_B * _S, _HD), lambda: (0, 0)),
        out_shape=jax.ShapeDtypeStruct((_B * _S, _HD), jnp.float32),
        scratch_shapes=[
            pltpu.VMEM((_NBUF, _CHUNK, _E), jnp.float32),
            pltpu.VMEM((_B, _BS, _E), jnp.float32),
            pltpu.SemaphoreType.DMA((_NBUF,)),
            pltpu.SemaphoreType.DMA((_B,)),
        ],
    )(xf, Wq, Wk, Wv)
    return out.reshape(_B, _S, _HD)


# speedup vs baseline: 1.1716x; 1.1716x over previous
import jax
import jax.numpy as jnp
from jax.experimental import pallas as pl
from jax.experimental.pallas import tpu as pltpu

_B = 3
_S = 2048
_E = 1024
_HD = 64
_BS = 16
_CHUNK = 1024
_NBUF = 4
_SCALE = _HD ** -0.5


def _paged_attn_kernel(x_hbm, wq_ref, wk_ref, wv_ref, out_ref,
                       xbuf, x16buf, sem, sem16):
    nchunks = _B * _S // _CHUNK
    chunks_per_req = _S // _CHUNK

    def copy(c):
        return pltpu.make_async_copy(
            x_hbm.at[pl.ds(c * _CHUNK, _CHUNK), :], xbuf.at[c % _NBUF],
            sem.at[c % _NBUF])

    def copy16(b):
        return pltpu.make_async_copy(
            x_hbm.at[pl.ds(b * _S, _BS), :], x16buf.at[b], sem16.at[b])

    for b in range(_B):
        copy16(b).start()
    for c in range(_NBUF):
        copy(c).start()

    dn_nt = (((1,), (1,)), ((), ()))
    dn_nn = (((1,), (0,)), ((), ()))

    a_req = [None] * _B
    v_req = [None] * _B
    for c in range(nchunks):
        b = c // chunks_per_req
        if c % chunks_per_req == 0:
            copy16(b).wait()
            x16 = x16buf[b]
            k16 = jax.lax.dot_general(x16, wk_ref[:, :], dn_nt,
                                      preferred_element_type=jnp.float32)
            v_req[b] = jax.lax.dot_general(x16, wv_ref[:, :], dn_nt,
                                           preferred_element_type=jnp.float32)
            a_req[b] = jax.lax.dot_general(k16 * _SCALE, wq_ref[:, :], dn_nn,
                                           preferred_element_type=jnp.float32)

        copy(c).wait()
        x_tile = xbuf[c % _NBUF]

        s_t = jax.lax.dot_general(a_req[b], x_tile, dn_nt,
                                  preferred_element_type=jnp.float32)

        row = jax.lax.broadcasted_iota(jnp.int32, (_BS, _CHUNK), 1)
        m = jax.lax.broadcasted_iota(jnp.int32, (_BS, _CHUNK), 0)
        d = (c % chunks_per_req) * (_CHUNK // _BS) + (row >> 4)
        r = row & (_BS - 1)
        cnt = d.astype(jnp.float32) + (m <= r).astype(jnp.float32)

        smax = jnp.max(s_t, axis=0, keepdims=True)
        w = cnt * jnp.exp(s_t - smax)
        w = w / jnp.sum(w, axis=0, keepdims=True)

        out = jax.lax.dot_general(w, v_req[b],
                                  (((0,), (0,)), ((), ())),
                                  preferred_element_type=jnp.float32)
        out_ref[pl.ds(c * _CHUNK, _CHUNK), :] = out

        if c + _NBUF < nchunks:
            copy(c + _NBUF).start()


@jax.jit
def kernel(x, Wq, Wk, Wv):
    xf = x.reshape(_B * _S, _E)
    out = pl.pallas_call(
        _paged_attn_kernel,
        in_specs=[
            pl.BlockSpec(memory_space=pltpu.MemorySpace.HBM),
            pl.BlockSpec((_HD, _E), lambda: (0, 0)),
            pl.BlockSpec((_HD, _E), lambda: (0, 0)),
            pl.BlockSpec((_HD, _E), lambda: (0, 0)),
        ],
        out_specs=pl.BlockSpec((_B * _S, _HD), lambda: (0, 0)),
        out_shape=jax.ShapeDtypeStruct((_B * _S, _HD), jnp.float32),
        scratch_shapes=[
            pltpu.VMEM((_NBUF, _CHUNK, _E), jnp.float32),
            pltpu.VMEM((_B, _BS, _E), jnp.float32),
            pltpu.SemaphoreType.DMA((_NBUF,)),
            pltpu.SemaphoreType.DMA((_B,)),
        ],
    )(xf, Wq, Wk, Wv)
    return out.reshape(_B, _S, _HD)


# speedup vs baseline: 1.1792x; 1.0065x over previous
import jax
import jax.numpy as jnp
from jax.experimental import pallas as pl
from jax.experimental.pallas import tpu as pltpu

_B = 3
_S = 2048
_E = 1024
_HD = 64
_BS = 16
_CHUNK = 1024
_NBUF = 4
_SCALE = _HD ** -0.5

_PLAN = [(0, 1024), (1024, 1024), (2048, 1024), (3072, 1024),
         (4096, 1024), (5120, 512), (5632, 512)]


def _paged_attn_kernel(x_hbm, wq_ref, wk_ref, wv_ref, out_hbm,
                       xbuf, x16buf, obuf, sem, sem16, osem):
    nchunks = len(_PLAN)

    def copy(c):
        st, sz = _PLAN[c]
        return pltpu.make_async_copy(
            x_hbm.at[pl.ds(st, sz), :],
            xbuf.at[c % _NBUF, pl.ds(0, sz), :],
            sem.at[c % _NBUF])

    def copy16(b):
        return pltpu.make_async_copy(
            x_hbm.at[pl.ds(b * _S, _BS), :], x16buf.at[b], sem16.at[b])

    def ocopy(c):
        st, sz = _PLAN[c]
        return pltpu.make_async_copy(
            obuf.at[c % 2, pl.ds(0, sz), :],
            out_hbm.at[pl.ds(st, sz), :],
            osem.at[c % 2])

    for b in range(_B):
        copy16(b).start()
    for c in range(_NBUF):
        copy(c).start()

    dn_nt = (((1,), (1,)), ((), ()))
    dn_nn = (((1,), (0,)), ((), ()))

    a_req = [None] * _B
    v_req = [None] * _B
    for c in range(nchunks):
        st, sz = _PLAN[c]
        b = st // _S
        if st == b * _S:
            copy16(b).wait()
            x16 = x16buf[b]
            k16 = jax.lax.dot_general(x16, wk_ref[:, :], dn_nt,
                                      preferred_element_type=jnp.float32)
            v_req[b] = jax.lax.dot_general(x16, wv_ref[:, :], dn_nt,
                                           preferred_element_type=jnp.float32)
            a_req[b] = jax.lax.dot_general(k16 * _SCALE, wq_ref[:, :], dn_nn,
                                           preferred_element_type=jnp.float32)

        copy(c).wait()
        x_tile = xbuf[c % _NBUF, pl.ds(0, sz), :]

        s_t = jax.lax.dot_general(a_req[b], x_tile, dn_nt,
                                  preferred_element_type=jnp.float32)

        row = jax.lax.broadcasted_iota(jnp.int32, (_BS, sz), 1)
        m = jax.lax.broadcasted_iota(jnp.int32, (_BS, sz), 0)
        d = ((st - b * _S) >> 4) + (row >> 4)
        r = row & (_BS - 1)
        cnt = d.astype(jnp.float32) + (m <= r).astype(jnp.float32)

        smax = jnp.max(s_t, axis=0, keepdims=True)
        w = cnt * jnp.exp(s_t - smax)
        w = w / jnp.sum(w, axis=0, keepdims=True)

        out = jax.lax.dot_general(w, v_req[b],
                                  (((0,), (0,)), ((), ())),
                                  preferred_element_type=jnp.float32)

        if c >= 2:
            ocopy(c - 2).wait()
        obuf[c % 2, pl.ds(0, sz), :] = out
        ocopy(c).start()

        if c + _NBUF < nchunks:
            copy(c + _NBUF).start()

    ocopy(nchunks - 2).wait()
    ocopy(nchunks - 1).wait()


@jax.jit
def kernel(x, Wq, Wk, Wv):
    xf = x.reshape(_B * _S, _E)
    out = pl.pallas_call(
        _paged_attn_kernel,
        in_specs=[
            pl.BlockSpec(memory_space=pltpu.MemorySpace.HBM),
            pl.BlockSpec((_HD, _E), lambda: (0, 0)),
            pl.BlockSpec((_HD, _E), lambda: (0, 0)),
            pl.BlockSpec((_HD, _E), lambda: (0, 0)),
        ],
        out_specs=pl.BlockSpec(memory_space=pltpu.MemorySpace.HBM),
        out_shape=jax.ShapeDtypeStruct((_B * _S, _HD), jnp.float32),
        scratch_shapes=[
            pltpu.VMEM((_NBUF, _CHUNK, _E), jnp.float32),
            pltpu.VMEM((_B, _BS, _E), jnp.float32),
            pltpu.VMEM((2, _CHUNK, _HD), jnp.float32),
            pltpu.SemaphoreType.DMA((_NBUF,)),
            pltpu.SemaphoreType.DMA((_B,)),
            pltpu.SemaphoreType.DMA((2,)),
        ],
    )(xf, Wq, Wk, Wv)
    return out.reshape(_B, _S, _HD)
